# baseline (speedup 1.0000x reference)
import functools

import jax
import jax.numpy as jnp
from jax import lax
from jax.experimental import pallas as pl
from jax.experimental.pallas import tpu as pltpu
from jax.experimental.pallas import tpu_sc as plsc

N_NODES = 10000
N_EDGES = 320000
IN_DIM = 128
HID = 16
OUT_DIM = 7

NC = 2
NS = 16
NW = NC * NS
LANES = 16
CHUNK = 128
G = 25
IDXW = 400
N_PAD = 10240
RPS = N_PAD // NS

_sc_mesh = plsc.VectorSubcoreMesh(core_axis_name="c", subcore_axis_name="s")
_sc_params = pltpu.CompilerParams(use_tc_tiling_on_sc=False)


def _zero_acc_slice(buf_v, acc, sid):
    zero = jnp.zeros((LANES,), jnp.float32)

    @pl.loop(0, CHUNK)
    def _(i):
        buf_v[i, :] = zero

    @pl.loop(0, RPS // CHUNK)
    def _(j):
        pltpu.sync_copy(buf_v, acc.at[pl.ds(sid * RPS + j * CHUNK, CHUNK)])


def _count_body(dst_hbm, out_hbm, buf_v, obuf, didx_v, sem, acc):
    cid = lax.axis_index("c")
    sid = lax.axis_index("s")
    gw = cid * NS + sid

    _zero_acc_slice(buf_v, acc, sid)
    one = jnp.ones((LANES,), jnp.float32)

    @pl.loop(0, IDXW)
    def _(i):
        obuf[i, :] = one

    pltpu.sync_copy(dst_hbm.at[gw], didx_v)
    plsc.subcore_barrier()

    for g in range(G):
        pltpu.async_copy(obuf, acc.at[didx_v.at[g]], sem, add=True)
    for g in range(G):
        pltpu.make_async_copy(obuf, acc.at[didx_v.at[g]], sem).wait()

    plsc.subcore_barrier()
    pltpu.sync_copy(acc.at[pl.ds(sid * RPS, RPS)],
                    out_hbm.at[cid].at[pl.ds(sid * RPS, RPS)])


def _ring_agg(hsp, acc, sidx_v, didx_v, bufs, gsems, ssems):
    pltpu.async_copy(hsp.at[sidx_v.at[0]], bufs[0], gsems[0])
    pltpu.async_copy(hsp.at[sidx_v.at[1]], bufs[1], gsems[1])
    for g in range(G):
        b = g % 4
        pltpu.make_async_copy(hsp.at[sidx_v.at[g]], bufs[b], gsems[b]).wait()
        pltpu.async_copy(bufs[b], acc.at[didx_v.at[g]], ssems[b], add=True)
        gf = g + 2
        if gf < G:
            bf = gf % 4
            if gf >= 4:
                pltpu.make_async_copy(
                    bufs[bf], acc.at[didx_v.at[gf - 4]], ssems[bf]).wait()
            pltpu.async_copy(hsp.at[sidx_v.at[gf]], bufs[bf], gsems[bf])
    for g in range(G - 4, G):
        b = g % 4
        pltpu.make_async_copy(bufs[b], acc.at[didx_v.at[g]], ssems[b]).wait()


def _agg_body(h_hbm, src_hbm, dst_hbm, out_hbm,
              buf_v, sidx_v, didx_v, bufs, gsems, ssems, hsp, acc):
    cid = lax.axis_index("c")
    sid = lax.axis_index("s")
    gw = cid * NS + sid

    _zero_acc_slice(buf_v, acc, sid)
    pltpu.sync_copy(h_hbm.at[pl.ds(sid * RPS, RPS)],
                    hsp.at[pl.ds(sid * RPS, RPS)])
    pltpu.sync_copy(src_hbm.at[gw], sidx_v)
    pltpu.sync_copy(dst_hbm.at[gw], didx_v)
    plsc.subcore_barrier()

    _ring_agg(hsp, acc, sidx_v, didx_v, bufs, gsems, ssems)

    plsc.subcore_barrier()
    pltpu.sync_copy(acc.at[pl.ds(sid * RPS, RPS)],
                    out_hbm.at[cid].at[pl.ds(sid * RPS, RPS)])


@functools.partial(
    pl.kernel,
    out_type=jax.ShapeDtypeStruct((NC, N_PAD, LANES), jnp.float32),
    mesh=_sc_mesh,
    scratch_types=[
        pltpu.VMEM((CHUNK, LANES), jnp.float32),
        pltpu.VMEM((IDXW, LANES), jnp.float32),
        pltpu.VMEM((G, IDXW), jnp.int32),
        pltpu.SemaphoreType.DMA,
        pltpu.VMEM_SHARED((N_PAD, LANES), jnp.float32),
    ],
    compiler_params=_sc_params,
)
def _sc_count(dst_hbm, out_hbm, buf_v, obuf, didx_v, sem, acc):
    _count_body(dst_hbm, out_hbm, buf_v, obuf, didx_v, sem, acc)


@functools.partial(
    pl.kernel,
    out_type=jax.ShapeDtypeStruct((NC, N_PAD, LANES), jnp.float32),
    mesh=_sc_mesh,
    scratch_types=(
        [pltpu.VMEM((CHUNK, LANES), jnp.float32),
         pltpu.VMEM((G, IDXW), jnp.int32),
         pltpu.VMEM((G, IDXW), jnp.int32)]
        + [pltpu.VMEM((IDXW, LANES), jnp.float32)] * 4
        + [pltpu.SemaphoreType.DMA] * 8
        + [pltpu.VMEM_SHARED((N_PAD, LANES), jnp.float32)] * 2
    ),
    compiler_params=_sc_params,
)
def _sc_agg(h_hbm, src_hbm, dst_hbm, out_hbm,
            buf_v, sidx_v, didx_v, *rest):
    bufs = rest[:4]
    gsems = rest[4:8]
    ssems = rest[8:12]
    hsp, acc = rest[12], rest[13]
    _agg_body(h_hbm, src_hbm, dst_hbm, out_hbm,
              buf_v, sidx_v, didx_v, bufs, gsems, ssems, hsp, acc)


@functools.partial(
    pl.kernel,
    out_type=[jax.ShapeDtypeStruct((NC, N_PAD, LANES), jnp.float32),
              jax.ShapeDtypeStruct((NC, N_PAD, LANES), jnp.float32),
              jax.ShapeDtypeStruct((NC, N_PAD, LANES), jnp.float32)],
    mesh=_sc_mesh,
    scratch_types=(
        [pltpu.VMEM((CHUNK, LANES), jnp.float32),
         pltpu.VMEM((G, IDXW), jnp.int32),
         pltpu.VMEM((G, IDXW), jnp.int32)]
        + [pltpu.VMEM((IDXW, LANES), jnp.float32)] * 4
        + [pltpu.VMEM((RPS, LANES), jnp.float32)] * 3
        + [pltpu.SemaphoreType.DMA] * 8
        + [pltpu.VMEM_SHARED((N_PAD, LANES), jnp.float32)] * 2
    ),
    compiler_params=_sc_params,
)
def _sc_agg1(cnt_hbm, h1_hbm, src_hbm, dst_hbm, p_hbm, h1p_hbm, dinv_hbm,
             buf_v, sidx_v, didx_v, gb0, gb1, gb2, gb3, c0, c1, hp,
             gs0, gs1, gs2, gs3, ss0, ss1, ss2, ss3, hsp, acc):
    cid = lax.axis_index("c")
    sid = lax.axis_index("s")
    gw = cid * NS + sid
    rs = pl.ds(sid * RPS, RPS)

    _zero_acc_slice(buf_v, acc, sid)
    pltpu.sync_copy(cnt_hbm.at[0].at[rs], c0)
    pltpu.sync_copy(cnt_hbm.at[1].at[rs], c1)
    pltpu.sync_copy(h1_hbm.at[rs], hp)
    pltpu.sync_copy(src_hbm.at[gw], sidx_v)
    pltpu.sync_copy(dst_hbm.at[gw], didx_v)

    @pl.loop(0, RPS)
    def _(i):
        x = c0[i, :] + c1[i, :] + 1.0
        bits = lax.bitcast_convert_type(x, jnp.int32)
        bits = 0x5F3759DF - lax.shift_right_logical(bits, 1)
        y = lax.bitcast_convert_type(bits, jnp.float32)
        y = y * (1.5 - 0.5 * x * y * y)
        y = y * (1.5 - 0.5 * x * y * y)
        y = y * (1.5 - 0.5 * x * y * y)
        c0[i, :] = y
        hp[i, :] = hp[i, :] * y

    pltpu.sync_copy(hp, hsp.at[rs])
    pltpu.sync_copy(hp, h1p_hbm.at[cid].at[rs])
    pltpu.sync_copy(c0, dinv_hbm.at[cid].at[rs])
    plsc.subcore_barrier()

    _ring_agg(hsp, acc, sidx_v, didx_v,
              (gb0, gb1, gb2, gb3), (gs0, gs1, gs2, gs3),
              (ss0, ss1, ss2, ss3))

    plsc.subcore_barrier()
    pltpu.sync_copy(acc.at[rs], p_hbm.at[cid].at[rs])


MM_BLK = 2048
ROW_BLK = 2048


def _mm_body(x_ref, w_ref, o_ref):
    o_ref[...] = jnp.dot(x_ref[...], w_ref[...],
                         preferred_element_type=jnp.float32)


def _tc_matmul(x, W1):
    return pl.pallas_call(
        _mm_body,
        grid=(N_PAD // MM_BLK,),
        in_specs=[pl.BlockSpec((MM_BLK, IN_DIM), lambda i: (i, 0)),
                  pl.BlockSpec((IN_DIM, HID), lambda i: (0, 0))],
        out_specs=pl.BlockSpec((MM_BLK, HID), lambda i: (i, 0)),
        out_shape=jax.ShapeDtypeStruct((N_PAD, HID), jnp.float32),
    )(x, W1)


def _mid_body(p_ref, hp_ref, dinv_ref, w2_ref, b1_ref, h2p_ref):
    agg = p_ref[0] + p_ref[1] + hp_ref[0]
    dv = dinv_ref[0]
    z = jnp.maximum(dv * agg + b1_ref[...], 0.0)
    h2 = jnp.dot(z, w2_ref[...], preferred_element_type=jnp.float32)
    h2c = h2 * dv[:, :OUT_DIM]
    h2p_ref[...] = jnp.concatenate(
        [h2c, jnp.zeros((h2c.shape[0], LANES - OUT_DIM), jnp.float32)],
        axis=1)


def _tc_mid(p, h1p, dinv, W2, b1):
    return pl.pallas_call(
        _mid_body,
        grid=(N_PAD // ROW_BLK,),
        in_specs=[pl.BlockSpec((NC, ROW_BLK, LANES), lambda i: (0, i, 0)),
                  pl.BlockSpec((1, ROW_BLK, LANES), lambda i: (0, i, 0)),
                  pl.BlockSpec((1, ROW_BLK, LANES), lambda i: (0, i, 0)),
                  pl.BlockSpec((HID, OUT_DIM), lambda i: (0, 0)),
                  pl.BlockSpec((LANES,), lambda i: (0,))],
        out_specs=pl.BlockSpec((ROW_BLK, LANES), lambda i: (i, 0)),
        out_shape=jax.ShapeDtypeStruct((N_PAD, LANES), jnp.float32),
    )(p, h1p, dinv, W2, b1)


def _out_body(q_ref, h2p_ref, dinv_ref, b2_ref, o_ref):
    s = (q_ref[0] + q_ref[1] + h2p_ref[...])[:, :OUT_DIM]
    o = dinv_ref[0][:, :OUT_DIM] * s + b2_ref[...]
    m = jnp.max(o, axis=1, keepdims=True)
    e = jnp.exp(o - m)
    lse = jnp.log(jnp.sum(e, axis=1, keepdims=True)) + m
    o_ref[...] = o - lse


def _tc_out(q, h2p, dinv, b2):
    return pl.pallas_call(
        _out_body,
        grid=(N_PAD // ROW_BLK,),
        in_specs=[pl.BlockSpec((NC, ROW_BLK, LANES), lambda i: (0, i, 0)),
                  pl.BlockSpec((ROW_BLK, LANES), lambda i: (i, 0)),
                  pl.BlockSpec((1, ROW_BLK, LANES), lambda i: (0, i, 0)),
                  pl.BlockSpec((OUT_DIM,), lambda i: (0,))],
        out_specs=pl.BlockSpec((ROW_BLK, OUT_DIM), lambda i: (i, 0)),
        out_shape=jax.ShapeDtypeStruct((N_PAD, OUT_DIM), jnp.float32),
    )(q, h2p, dinv, b2)


@jax.jit
def kernel(x, edge_index, W1, b1, W2, b2):
    ei = edge_index.astype(jnp.int32)
    src = ei[0].reshape(NW, G, IDXW)
    dst = ei[1].reshape(NW, G, IDXW)

    cnt = _sc_count(dst)
    h1 = _tc_matmul(x, W1)

    p, h1p, dinv = _sc_agg1(cnt, h1, src, dst)
    h2p = _tc_mid(p, h1p, dinv, W2, b1)

    q = _sc_agg(h2p, src, dst)
    out = _tc_out(q, h2p, dinv, b2)
    return out[:N_NODES]


# speedup vs baseline: 1.0394x; 1.0394x over previous
import functools

import jax
import jax.numpy as jnp
from jax import lax
from jax.experimental import pallas as pl
from jax.experimental.pallas import tpu as pltpu
from jax.experimental.pallas import tpu_sc as plsc

N_NODES = 10000
N_EDGES = 320000
IN_DIM = 128
HID = 16
OUT_DIM = 7

NC = 2
NS = 16
NW = NC * NS
LANES = 16
CHUNK = 128
G = 10
IDXW = 1000
N_PAD = 10240
RPS = N_PAD // NS

_sc_mesh = plsc.VectorSubcoreMesh(core_axis_name="c", subcore_axis_name="s")
_sc_params = pltpu.CompilerParams(use_tc_tiling_on_sc=False)


def _zero_acc_slice(buf_v, acc, sid):
    zero = jnp.zeros((LANES,), jnp.float32)

    @pl.loop(0, CHUNK)
    def _(i):
        buf_v[i, :] = zero

    @pl.loop(0, RPS // CHUNK)
    def _(j):
        pltpu.sync_copy(buf_v, acc.at[pl.ds(sid * RPS + j * CHUNK, CHUNK)])


def _count_body(dst_hbm, out_hbm, buf_v, obuf, didx_v, sem, acc):
    cid = lax.axis_index("c")
    sid = lax.axis_index("s")
    gw = cid * NS + sid

    _zero_acc_slice(buf_v, acc, sid)
    one = jnp.ones((LANES,), jnp.float32)

    @pl.loop(0, IDXW)
    def _(i):
        obuf[i, :] = one

    pltpu.sync_copy(dst_hbm.at[gw], didx_v)
    plsc.subcore_barrier()

    for g in range(G):
        pltpu.async_copy(obuf, acc.at[didx_v.at[g]], sem, add=True)
    for g in range(G):
        pltpu.make_async_copy(obuf, acc.at[didx_v.at[g]], sem).wait()

    plsc.subcore_barrier()
    pltpu.sync_copy(acc.at[pl.ds(sid * RPS, RPS)],
                    out_hbm.at[cid].at[pl.ds(sid * RPS, RPS)])


def _agg_body(h_hbm, src_hbm, dst_hbm, out_hbm,
              buf_v, sidx_v, didx_v, gbuf0, gbuf1, sem0, sem1, hsp, acc):
    cid = lax.axis_index("c")
    sid = lax.axis_index("s")
    gw = cid * NS + sid

    _zero_acc_slice(buf_v, acc, sid)
    pltpu.sync_copy(h_hbm.at[pl.ds(sid * RPS, RPS)],
                    hsp.at[pl.ds(sid * RPS, RPS)])
    pltpu.sync_copy(src_hbm.at[gw], sidx_v)
    pltpu.sync_copy(dst_hbm.at[gw], didx_v)
    plsc.subcore_barrier()

    gbufs = (gbuf0, gbuf1)
    sems = (sem0, sem1)
    pltpu.async_copy(hsp.at[sidx_v.at[0]], gbuf0, sem0)
    pltpu.async_copy(hsp.at[sidx_v.at[1]], gbuf1, sem1)
    for g in range(G):
        b = g % 2
        pltpu.make_async_copy(
            hsp.at[sidx_v.at[g]], gbufs[b], sems[b]).wait()
        pltpu.sync_copy(gbufs[b], acc.at[didx_v.at[g]], add=True)
        if g + 2 < G:
            pltpu.async_copy(
                hsp.at[sidx_v.at[g + 2]], gbufs[b], sems[b])

    plsc.subcore_barrier()
    pltpu.sync_copy(acc.at[pl.ds(sid * RPS, RPS)],
                    out_hbm.at[cid].at[pl.ds(sid * RPS, RPS)])


@functools.partial(
    pl.kernel,
    out_type=jax.ShapeDtypeStruct((NC, N_PAD, LANES), jnp.float32),
    mesh=_sc_mesh,
    scratch_types=[
        pltpu.VMEM((CHUNK, LANES), jnp.float32),
        pltpu.VMEM((IDXW, LANES), jnp.float32),
        pltpu.VMEM((G, IDXW), jnp.int32),
        pltpu.SemaphoreType.DMA,
        pltpu.VMEM_SHARED((N_PAD, LANES), jnp.float32),
    ],
    compiler_params=_sc_params,
)
def _sc_count(dst_hbm, out_hbm, buf_v, obuf, didx_v, sem, acc):
    _count_body(dst_hbm, out_hbm, buf_v, obuf, didx_v, sem, acc)


@functools.partial(
    pl.kernel,
    out_type=jax.ShapeDtypeStruct((NC, N_PAD, LANES), jnp.float32),
    mesh=_sc_mesh,
    scratch_types=(
        [pltpu.VMEM((CHUNK, LANES), jnp.float32),
         pltpu.VMEM((G, IDXW), jnp.int32),
         pltpu.VMEM((G, IDXW), jnp.int32)]
        + [pltpu.VMEM((IDXW, LANES), jnp.float32)] * 2
        + [pltpu.SemaphoreType.DMA] * 2
        + [pltpu.VMEM_SHARED((N_PAD, LANES), jnp.float32)] * 2
    ),
    compiler_params=_sc_params,
)
def _sc_agg(h_hbm, src_hbm, dst_hbm, out_hbm,
            buf_v, sidx_v, didx_v, gbuf0, gbuf1, sem0, sem1, hsp, acc):
    _agg_body(h_hbm, src_hbm, dst_hbm, out_hbm,
              buf_v, sidx_v, didx_v, gbuf0, gbuf1, sem0, sem1, hsp, acc)


@functools.partial(
    pl.kernel,
    out_type=[jax.ShapeDtypeStruct((NC, N_PAD, LANES), jnp.float32),
              jax.ShapeDtypeStruct((NC, N_PAD, LANES), jnp.float32),
              jax.ShapeDtypeStruct((NC, N_PAD, LANES), jnp.float32)],
    mesh=_sc_mesh,
    scratch_types=(
        [pltpu.VMEM((CHUNK, LANES), jnp.float32),
         pltpu.VMEM((G, IDXW), jnp.int32),
         pltpu.VMEM((G, IDXW), jnp.int32)]
        + [pltpu.VMEM((IDXW, LANES), jnp.float32)] * 2
        + [pltpu.VMEM((RPS, LANES), jnp.float32)] * 4
        + [pltpu.SemaphoreType.DMA] * 2
        + [pltpu.VMEM_SHARED((N_PAD, LANES), jnp.float32)] * 2
    ),
    compiler_params=_sc_params,
)
def _sc_agg1(cnt_hbm, h1_hbm, src_hbm, dst_hbm, p_hbm, h1p_hbm, dinv_hbm,
             buf_v, sidx_v, didx_v, gbuf0, gbuf1, c0, c1, hp, dv,
             sem0, sem1, hsp, acc):
    cid = lax.axis_index("c")
    sid = lax.axis_index("s")
    gw = cid * NS + sid
    rs = pl.ds(sid * RPS, RPS)

    _zero_acc_slice(buf_v, acc, sid)
    pltpu.sync_copy(cnt_hbm.at[0].at[rs], c0)
    pltpu.sync_copy(cnt_hbm.at[1].at[rs], c1)
    pltpu.sync_copy(h1_hbm.at[rs], hp)
    pltpu.sync_copy(src_hbm.at[gw], sidx_v)
    pltpu.sync_copy(dst_hbm.at[gw], didx_v)

    @pl.loop(0, RPS)
    def _(i):
        x = c0[i, :] + c1[i, :] + 1.0
        bits = lax.bitcast_convert_type(x, jnp.int32)
        bits = 0x5F3759DF - lax.shift_right_logical(bits, 1)
        y = lax.bitcast_convert_type(bits, jnp.float32)
        y = y * (1.5 - 0.5 * x * y * y)
        y = y * (1.5 - 0.5 * x * y * y)
        y = y * (1.5 - 0.5 * x * y * y)
        dv[i, :] = y
        hp[i, :] = hp[i, :] * y

    pltpu.sync_copy(hp, hsp.at[rs])
    pltpu.sync_copy(hp, h1p_hbm.at[cid].at[rs])
    pltpu.sync_copy(dv, dinv_hbm.at[cid].at[rs])
    plsc.subcore_barrier()

    gbufs = (gbuf0, gbuf1)
    sems = (sem0, sem1)
    pltpu.async_copy(hsp.at[sidx_v.at[0]], gbuf0, sem0)
    pltpu.async_copy(hsp.at[sidx_v.at[1]], gbuf1, sem1)
    for g in range(G):
        b = g % 2
        pltpu.make_async_copy(
            hsp.at[sidx_v.at[g]], gbufs[b], sems[b]).wait()
        pltpu.sync_copy(gbufs[b], acc.at[didx_v.at[g]], add=True)
        if g + 2 < G:
            pltpu.async_copy(
                hsp.at[sidx_v.at[g + 2]], gbufs[b], sems[b])

    plsc.subcore_barrier()
    pltpu.sync_copy(acc.at[rs], p_hbm.at[cid].at[rs])


MM_BLK = 2048
ROW_BLK = 2048


def _mm_body(x_ref, w_ref, o_ref):
    o_ref[...] = jnp.dot(x_ref[...], w_ref[...],
                         preferred_element_type=jnp.float32)


def _tc_matmul(x, W1):
    return pl.pallas_call(
        _mm_body,
        grid=(N_PAD // MM_BLK,),
        in_specs=[pl.BlockSpec((MM_BLK, IN_DIM), lambda i: (i, 0)),
                  pl.BlockSpec((IN_DIM, HID), lambda i: (0, 0))],
        out_specs=pl.BlockSpec((MM_BLK, HID), lambda i: (i, 0)),
        out_shape=jax.ShapeDtypeStruct((N_PAD, HID), jnp.float32),
    )(x, W1)


def _mid_body(p_ref, hp_ref, dinv_ref, w2_ref, b1_ref, h2p_ref):
    agg = p_ref[0] + p_ref[1] + hp_ref[0]
    dv = dinv_ref[0]
    z = jnp.maximum(dv * agg + b1_ref[...], 0.0)
    h2 = jnp.dot(z, w2_ref[...], preferred_element_type=jnp.float32)
    h2c = h2 * dv[:, :OUT_DIM]
    h2p_ref[...] = jnp.concatenate(
        [h2c, jnp.zeros((h2c.shape[0], LANES - OUT_DIM), jnp.float32)],
        axis=1)


def _tc_mid(p, h1p, dinv, W2, b1):
    return pl.pallas_call(
        _mid_body,
        grid=(N_PAD // ROW_BLK,),
        in_specs=[pl.BlockSpec((NC, ROW_BLK, LANES), lambda i: (0, i, 0)),
                  pl.BlockSpec((1, ROW_BLK, LANES), lambda i: (0, i, 0)),
                  pl.BlockSpec((1, ROW_BLK, LANES), lambda i: (0, i, 0)),
                  pl.BlockSpec((HID, OUT_DIM), lambda i: (0, 0)),
                  pl.BlockSpec((LANES,), lambda i: (0,))],
        out_specs=pl.BlockSpec((ROW_BLK, LANES), lambda i: (i, 0)),
        out_shape=jax.ShapeDtypeStruct((N_PAD, LANES), jnp.float32),
    )(p, h1p, dinv, W2, b1)


def _out_body(q_ref, h2p_ref, dinv_ref, b2_ref, o_ref):
    s = (q_ref[0] + q_ref[1] + h2p_ref[...])[:, :OUT_DIM]
    o = dinv_ref[0][:, :OUT_DIM] * s + b2_ref[...]
    m = jnp.max(o, axis=1, keepdims=True)
    e = jnp.exp(o - m)
    lse = jnp.log(jnp.sum(e, axis=1, keepdims=True)) + m
    o_ref[...] = o - lse


def _tc_out(q, h2p, dinv, b2):
    return pl.pallas_call(
        _out_body,
        grid=(N_PAD // ROW_BLK,),
        in_specs=[pl.BlockSpec((NC, ROW_BLK, LANES), lambda i: (0, i, 0)),
                  pl.BlockSpec((ROW_BLK, LANES), lambda i: (i, 0)),
                  pl.BlockSpec((1, ROW_BLK, LANES), lambda i: (0, i, 0)),
                  pl.BlockSpec((OUT_DIM,), lambda i: (0,))],
        out_specs=pl.BlockSpec((ROW_BLK, OUT_DIM), lambda i: (i, 0)),
        out_shape=jax.ShapeDtypeStruct((N_PAD, OUT_DIM), jnp.float32),
    )(q, h2p, dinv, b2)


@jax.jit
def kernel(x, edge_index, W1, b1, W2, b2):
    ei = edge_index.astype(jnp.int32)
    src = ei[0].reshape(NW, G, IDXW)
    dst = ei[1].reshape(NW, G, IDXW)

    cnt = _sc_count(dst)
    h1 = _tc_matmul(x, W1)

    p, h1p, dinv = _sc_agg1(cnt, h1, src, dst)
    h2p = _tc_mid(p, h1p, dinv, W2, b1)

    q = _sc_agg(h2p, src, dst)
    out = _tc_out(q, h2p, dinv, b2)
    return out[:N_NODES]
